# Initial kernel scaffold; baseline (speedup 1.0000x reference)
import jax
import jax.numpy as jnp
from jax.experimental import pallas as pl


def kernel(x, W_enc, b_enc, dec_bias, topk):
    raise NotImplementedError("write your pallas kernel here")


# speedup vs baseline: 15.7447x; 15.7447x over previous
import functools

import jax
import jax.numpy as jnp
from jax.experimental import pallas as pl
from jax.experimental.pallas import tpu as pltpu

K_STATIC = 64


def _encode_kernel(x_ref, w_ref, b_ref, z_ref):
    z = jax.lax.dot_general(
        x_ref[...], w_ref[...], (((1,), (1,)), ((), ())),
        preferred_element_type=jnp.float32)
    z_ref[...] = jnp.maximum(z + b_ref[...], 0.0)


def _select_kernel(k_ref, z_ref, t_ref, lo_ref, hi_ref, tb_ref):
    k = k_ref[0]
    bits0 = jax.lax.bitcast_convert_type(z_ref[...], jnp.int32)
    rowmax = jnp.max(bits0, axis=1, keepdims=True)

    lo_ref[...] = jnp.zeros_like(rowmax)
    hi_ref[...] = rowmax + 1
    tb_ref[...] = jnp.full_like(rowmax, -1)

    def cond(done):
        return jnp.logical_not(done)

    def body(done):
        lo, hi, tb = lo_ref[...], hi_ref[...], tb_ref[...]
        found = tb >= 0
        mid = jax.lax.shift_right_logical(lo + hi, 1)
        bits = jax.lax.bitcast_convert_type(z_ref[...], jnp.int32)
        cnt = jnp.sum((bits >= mid).astype(jnp.int32), axis=1, keepdims=True)
        upd = jnp.logical_not(found)
        ge = cnt >= k
        lo = jnp.where(jnp.logical_and(upd, ge), mid, lo)
        hi = jnp.where(jnp.logical_and(upd, jnp.logical_not(ge)), mid, hi)
        hit = jnp.logical_and(upd, cnt == k)
        closed = jnp.logical_and(upd, (hi - lo) <= 1)
        tb = jnp.where(hit, mid, jnp.where(closed, lo, tb))
        lo_ref[...] = lo
        hi_ref[...] = hi
        tb_ref[...] = tb
        return jnp.all(tb >= 0)

    jax.lax.while_loop(cond, body, jnp.bool_(False))
    t_ref[...] = jnp.where(k <= 0, jnp.int32(0x7F800000), tb_ref[...])


def _mask_decode_kernel(t_ref, z_ref, w_ref, db_ref, zm_ref, xhat_ref,
                        *, nsteps):
    i = pl.program_id(0)
    z = z_ref[...]
    bits = jax.lax.bitcast_convert_type(z, jnp.int32)
    zm = jnp.where(bits >= t_ref[...], z, 0.0)
    zm_ref[...] = zm
    part = jax.lax.dot_general(
        zm, w_ref[...], (((1,), (0,)), ((), ())),
        preferred_element_type=jnp.float32)

    @pl.when(i == 0)
    def _():
        xhat_ref[...] = part + db_ref[...]

    @pl.when(i != 0)
    def _():
        xhat_ref[...] = xhat_ref[...] + part

    del nsteps


def kernel(x, W_enc, b_enc, dec_bias, topk):
    B, d_in = x.shape
    width = W_enc.shape[0]
    wb1 = min(512, width)
    rb = min(128, B)
    wb2 = min(256, width)

    b2 = b_enc.reshape(1, width)
    db2 = dec_bias.reshape(1, d_in)
    k_eff = jnp.clip(jnp.asarray(topk, jnp.int32), 0, K_STATIC).reshape(1)

    z = pl.pallas_call(
        _encode_kernel,
        grid=(width // wb1,),
        in_specs=[
            pl.BlockSpec((B, d_in), lambda i: (0, 0)),
            pl.BlockSpec((wb1, d_in), lambda i: (i, 0)),
            pl.BlockSpec((1, wb1), lambda i: (0, i)),
        ],
        out_specs=pl.BlockSpec((B, wb1), lambda i: (0, i)),
        out_shape=jax.ShapeDtypeStruct((B, width), jnp.float32),
    )(x, W_enc, b2)

    t = pl.pallas_call(
        _select_kernel,
        grid_spec=pltpu.PrefetchScalarGridSpec(
            num_scalar_prefetch=1,
            grid=(B // rb,),
            in_specs=[pl.BlockSpec((rb, width), lambda i, k: (i, 0))],
            out_specs=pl.BlockSpec((rb, 1), lambda i, k: (i, 0)),
            scratch_shapes=[
                pltpu.VMEM((rb, 1), jnp.int32),
                pltpu.VMEM((rb, 1), jnp.int32),
                pltpu.VMEM((rb, 1), jnp.int32),
            ],
        ),
        out_shape=jax.ShapeDtypeStruct((B, 1), jnp.int32),
    )(k_eff, z)

    nsteps = width // wb2
    zm, xhat = pl.pallas_call(
        functools.partial(_mask_decode_kernel, nsteps=nsteps),
        grid=(nsteps,),
        in_specs=[
            pl.BlockSpec((B, 1), lambda i: (0, 0)),
            pl.BlockSpec((B, wb2), lambda i: (0, i)),
            pl.BlockSpec((wb2, d_in), lambda i: (i, 0)),
            pl.BlockSpec((1, d_in), lambda i: (0, 0)),
        ],
        out_specs=[
            pl.BlockSpec((B, wb2), lambda i: (0, i)),
            pl.BlockSpec((B, d_in), lambda i: (0, 0)),
        ],
        out_shape=[
            jax.ShapeDtypeStruct((B, width), jnp.float32),
            jax.ShapeDtypeStruct((B, d_in), jnp.float32),
        ],
    )(t, z, W_enc, db2)

    return (zm, xhat)


# speedup vs baseline: 17.9439x; 1.1397x over previous
import functools

import jax
import jax.numpy as jnp
from jax.experimental import pallas as pl
from jax.experimental.pallas import tpu as pltpu

K_STATIC = 64


def _encode_kernel(x_ref, w_ref, b_ref, z_ref):
    z = jax.lax.dot_general(
        x_ref[...], w_ref[...], (((1,), (1,)), ((), ())),
        preferred_element_type=jnp.float32)
    z_ref[...] = jnp.maximum(z + b_ref[...], 0.0)


def _select_kernel(k_ref, z_ref, t_ref, lo_ref, hi_ref, tb_ref, cm_ref):
    k = k_ref[0]
    bits0 = jax.lax.bitcast_convert_type(z_ref[...], jnp.int32)
    rowmax = jnp.max(bits0, axis=1, keepdims=True)

    ncm = cm_ref.shape[1]
    nslab = bits0.shape[1] // ncm
    cm = bits0[:, :ncm]
    for a in range(1, nslab):
        cm = jnp.maximum(cm, bits0[:, a * ncm:(a + 1) * ncm])
    cm_ref[...] = cm

    lo_ref[...] = jnp.zeros_like(rowmax)
    hi_ref[...] = rowmax + 1

    def s1_body(_, carry):
        lo, hi = lo_ref[...], hi_ref[...]
        mid = jax.lax.shift_right_logical(lo + hi, 1)
        c = jnp.sum((cm_ref[...] >= mid).astype(jnp.int32), axis=1,
                    keepdims=True)
        ge = c >= k
        lo_ref[...] = jnp.where(ge, mid, lo)
        hi_ref[...] = jnp.where(ge, hi, mid)
        return carry

    jax.lax.fori_loop(0, 18, s1_body, 0)

    lo_ref[...] = jnp.minimum(lo_ref[...], rowmax)
    hi_ref[...] = rowmax + 1
    tb_ref[...] = jnp.full_like(rowmax, -1)

    def cond(done):
        return jnp.logical_not(done)

    def body(done):
        lo, hi, tb = lo_ref[...], hi_ref[...], tb_ref[...]
        found = tb >= 0
        mid = jax.lax.shift_right_logical(lo + hi, 1)
        bits = jax.lax.bitcast_convert_type(z_ref[...], jnp.int32)
        cnt = jnp.sum((bits >= mid).astype(jnp.int32), axis=1, keepdims=True)
        upd = jnp.logical_not(found)
        ge = cnt >= k
        lo = jnp.where(jnp.logical_and(upd, ge), mid, lo)
        hi = jnp.where(jnp.logical_and(upd, jnp.logical_not(ge)), mid, hi)
        hit = jnp.logical_and(upd, cnt == k)
        closed = jnp.logical_and(upd, (hi - lo) <= 1)
        tb = jnp.where(hit, mid, jnp.where(closed, lo, tb))
        lo_ref[...] = lo
        hi_ref[...] = hi
        tb_ref[...] = tb
        return jnp.all(tb >= 0)

    jax.lax.while_loop(cond, body, jnp.bool_(False))
    t_ref[...] = jnp.where(k <= 0, jnp.int32(0x7F800000), tb_ref[...])


def _mask_decode_kernel(t_ref, z_ref, w_ref, db_ref, zm_ref, xhat_ref,
                        *, nsteps):
    i = pl.program_id(0)
    z = z_ref[...]
    bits = jax.lax.bitcast_convert_type(z, jnp.int32)
    zm = jnp.where(bits >= t_ref[...], z, 0.0)
    zm_ref[...] = zm
    part = jax.lax.dot_general(
        zm, w_ref[...], (((1,), (0,)), ((), ())),
        preferred_element_type=jnp.float32)

    @pl.when(i == 0)
    def _():
        xhat_ref[...] = part + db_ref[...]

    @pl.when(i != 0)
    def _():
        xhat_ref[...] = xhat_ref[...] + part

    del nsteps


def kernel(x, W_enc, b_enc, dec_bias, topk):
    B, d_in = x.shape
    width = W_enc.shape[0]
    wb1 = min(512, width)
    rb = min(128, B)
    wb2 = min(256, width)

    b2 = b_enc.reshape(1, width)
    db2 = dec_bias.reshape(1, d_in)
    k_eff = jnp.clip(jnp.asarray(topk, jnp.int32), 0, K_STATIC).reshape(1)

    z = pl.pallas_call(
        _encode_kernel,
        grid=(width // wb1,),
        in_specs=[
            pl.BlockSpec((B, d_in), lambda i: (0, 0)),
            pl.BlockSpec((wb1, d_in), lambda i: (i, 0)),
            pl.BlockSpec((1, wb1), lambda i: (0, i)),
        ],
        out_specs=pl.BlockSpec((B, wb1), lambda i: (0, i)),
        out_shape=jax.ShapeDtypeStruct((B, width), jnp.float32),
    )(x, W_enc, b2)

    t = pl.pallas_call(
        _select_kernel,
        grid_spec=pltpu.PrefetchScalarGridSpec(
            num_scalar_prefetch=1,
            grid=(B // rb,),
            in_specs=[pl.BlockSpec((rb, width), lambda i, k: (i, 0))],
            out_specs=pl.BlockSpec((rb, 1), lambda i, k: (i, 0)),
            scratch_shapes=[
                pltpu.VMEM((rb, 1), jnp.int32),
                pltpu.VMEM((rb, 1), jnp.int32),
                pltpu.VMEM((rb, 1), jnp.int32),
                pltpu.VMEM((rb, 256), jnp.int32),
            ],
        ),
        out_shape=jax.ShapeDtypeStruct((B, 1), jnp.int32),
    )(k_eff, z)

    nsteps = width // wb2
    zm, xhat = pl.pallas_call(
        functools.partial(_mask_decode_kernel, nsteps=nsteps),
        grid=(nsteps,),
        in_specs=[
            pl.BlockSpec((B, 1), lambda i: (0, 0)),
            pl.BlockSpec((B, wb2), lambda i: (0, i)),
            pl.BlockSpec((wb2, d_in), lambda i: (i, 0)),
            pl.BlockSpec((1, d_in), lambda i: (0, 0)),
        ],
        out_specs=[
            pl.BlockSpec((B, wb2), lambda i: (0, i)),
            pl.BlockSpec((B, d_in), lambda i: (0, 0)),
        ],
        out_shape=[
            jax.ShapeDtypeStruct((B, width), jnp.float32),
            jax.ShapeDtypeStruct((B, d_in), jnp.float32),
        ],
    )(t, z, W_enc, db2)

    return (zm, xhat)
